# baseline (speedup 1.0000x reference)
import functools

import jax
import jax.numpy as jnp
from jax import lax
from jax.experimental import pallas as pl
from jax.experimental.pallas import tpu as pltpu
from jax.experimental.pallas import tpu_sc as plsc

_NC = 2
_NS = 16
_NW = _NC * _NS
_LANES = 16


@functools.lru_cache(maxsize=None)
def _make_kernel(L: int, B: int, V2: int, D: int):
    LG = 8
    CB = 128
    n_lg = L // LG
    n_cb = B // CB
    n_tasks = n_lg * n_cb
    assert n_tasks % _NW == 0
    t_per_w = n_tasks // _NW

    mesh = plsc.VectorSubcoreMesh(core_axis_name="c", subcore_axis_name="s")

    @functools.partial(
        pl.kernel,
        out_type=jax.ShapeDtypeStruct((L, D, B), jnp.float32),
        mesh=mesh,
        scratch_types=[
            pltpu.VMEM((LG, CB), jnp.int32),
            pltpu.VMEM((2, CB), jnp.int32),
            pltpu.VMEM((2, CB, 2 * D), jnp.float32),
            pltpu.VMEM((2, D, CB), jnp.float32),
            pltpu.SemaphoreType.DMA,
            pltpu.SemaphoreType.DMA,
            pltpu.SemaphoreType.DMA,
            pltpu.SemaphoreType.DMA,
            pltpu.SemaphoreType.DMA,
        ],
        compiler_params=pltpu.CompilerParams(
            use_tc_tiling_on_sc=True, needs_layout_passes=False
        ),
    )
    def embed_kernel(xt_hbm, r2_hbm, ot_hbm, xtb, rid, land, obuf, isem,
                     gs0, gs1, os0, os1):
        gs = [gs0, gs1]
        os_ = [os0, os1]
        wid = lax.axis_index("s") * _NC + lax.axis_index("c")
        jvec = lax.iota(jnp.int32, _LANES)

        def prep_and_fire(task, p):
            lg = task // n_cb
            cb = task - lg * n_cb
            return lg, cb

        def fire_gather(l, p):
            for g in range(CB // _LANES):
                v = xtb[l, pl.ds(g * _LANES, _LANES)]
                rid[p, pl.ds(g * _LANES, _LANES)] = lax.shift_right_logical(v, 1)
            pltpu.async_copy(r2_hbm.at[rid.at[p]], land.at[p], gs[p])

        def wait_gather(p):
            pltpu.make_async_copy(r2_hbm.at[rid.at[p]], land.at[p], gs[p]).wait()

        def transpose_block(l, p):
            land_p = land.at[p]
            for g in range(CB // _LANES):
                jv = jvec + (g * _LANES)
                odd = lax.bitwise_and(xtb[l, pl.ds(g * _LANES, _LANES)], 1) * D

                def dloop(d, carry):
                    vals = plsc.load_gather(land_p, [jv, odd + d])
                    obuf[p, d, pl.ds(g * _LANES, _LANES)] = vals
                    return carry

                lax.fori_loop(0, D, dloop, 0)

        def fire_store(lg, l, cb, p):
            pltpu.async_copy(
                obuf.at[p], ot_hbm.at[lg * LG + l, :, pl.ds(cb * CB, CB)], os_[p]
            )

        def wait_store(lg, l, cb, p):
            pltpu.make_async_copy(
                obuf.at[p], ot_hbm.at[lg * LG + l, :, pl.ds(cb * CB, CB)], os_[p]
            ).wait()

        def do_task(t, carry):
            task = wid * t_per_w + t
            lg = task // n_cb
            cb = task - lg * n_cb
            pltpu.async_copy(
                xt_hbm.at[pl.ds(lg * LG, LG), pl.ds(cb * CB, CB)], xtb, isem
            ).wait()
            fire_gather(0, 0)
            for l in range(LG):
                p = l % 2
                if l + 1 < LG:
                    fire_gather(l + 1, (l + 1) % 2)
                wait_gather(p)
                if l >= 2:
                    wait_store(lg, l - 2, cb, p)
                transpose_block(l, p)
                fire_store(lg, l, cb, p)
            wait_store(lg, LG - 2, cb, 0)
            wait_store(lg, LG - 1, cb, 1)
            return carry

        lax.fori_loop(0, t_per_w, do_task, 0)

    return embed_kernel


def kernel(x, table):
    B, L = x.shape
    V, D = table.shape
    r2 = table.reshape(V // 2, 2 * D)
    xt = x.T.astype(jnp.int32)
    ot = _make_kernel(L, B, V // 2, D)(xt, r2)
    return ot.transpose(2, 0, 1)
